# baseline (speedup 1.0000x reference)
import functools

import jax
from jax import lax
from jax.experimental import pallas as pl
from jax.experimental.pallas import tpu as pltpu
from jax.experimental.pallas import tpu_sc as plsc


def _make_sc_broadcast(batch, seq_len, d_model, dtype):
    info = plsc.get_sparse_core_info()
    num_workers = info.num_cores * info.num_subcores
    rows_per_worker = seq_len // num_workers
    max_chunk = 104
    chunks = []
    rem = rows_per_worker
    while rem > 0:
        c = min(max_chunk, rem)
        chunks.append(c)
        rem -= c
    buf_rows = max(chunks)

    mesh = plsc.VectorSubcoreMesh(core_axis_name="c", subcore_axis_name="s")

    @functools.partial(
        pl.kernel,
        mesh=mesh,
        out_type=jax.ShapeDtypeStruct((batch, seq_len, d_model), dtype),
        scratch_types=[
            pltpu.VMEM((buf_rows, d_model), dtype),
            pltpu.SemaphoreType.DMA,
            pltpu.SemaphoreType.DMA,
        ],
    )
    def sc_broadcast(table_hbm, out_hbm, buf, rsem, wsem):
        wid = lax.axis_index("s") * info.num_cores + lax.axis_index("c")
        base = wid * rows_per_worker

        r0 = 0
        for chunk in chunks:
            row = base + r0
            cbuf = buf.at[pl.ds(0, chunk)]
            pltpu.async_copy(table_hbm.at[pl.ds(row, chunk)], cbuf, rsem).wait()
            writes = [
                pltpu.async_copy(cbuf, out_hbm.at[b, pl.ds(row, chunk)], wsem)
                for b in range(batch)
            ]
            for h in writes:
                h.wait()
            r0 += chunk

    return sc_broadcast


def kernel(x, table):
    batch, seq_len, d_model = x.shape
    fn = _make_sc_broadcast(batch, seq_len, d_model, table.dtype)
    return fn(table)


# speedup vs baseline: 1.0022x; 1.0022x over previous
import functools

import jax
from jax import lax
from jax.experimental import pallas as pl
from jax.experimental.pallas import tpu as pltpu
from jax.experimental.pallas import tpu_sc as plsc


def _make_sc_broadcast(batch, seq_len, d_model, dtype):
    info = plsc.get_sparse_core_info()
    num_workers = info.num_cores * info.num_subcores
    rows_per_worker = seq_len // num_workers
    max_chunk = 120
    chunks = []
    rem = rows_per_worker
    while rem > 0:
        c = min(max_chunk, rem)
        chunks.append(c)
        rem -= c
    buf_rows = max(chunks)

    mesh = plsc.VectorSubcoreMesh(core_axis_name="c", subcore_axis_name="s")

    @functools.partial(
        pl.kernel,
        mesh=mesh,
        out_type=jax.ShapeDtypeStruct((batch, seq_len, d_model), dtype),
        scratch_types=[
            pltpu.VMEM((buf_rows, d_model), dtype),
            pltpu.SemaphoreType.DMA,
            pltpu.SemaphoreType.DMA,
        ],
    )
    def sc_broadcast(table_hbm, out_hbm, buf, rsem, wsem):
        wid = lax.axis_index("s") * info.num_cores + lax.axis_index("c")
        base = wid * rows_per_worker

        r0 = 0
        for chunk in chunks:
            row = base + r0
            cbuf = buf.at[pl.ds(0, chunk)]
            pltpu.async_copy(table_hbm.at[pl.ds(row, chunk)], cbuf, rsem).wait()
            writes = [
                pltpu.async_copy(cbuf, out_hbm.at[b, pl.ds(row, chunk)], wsem)
                for b in range(batch)
            ]
            for h in writes:
                h.wait()
            r0 += chunk

    return sc_broadcast


def kernel(x, table):
    batch, seq_len, d_model = x.shape
    fn = _make_sc_broadcast(batch, seq_len, d_model, table.dtype)
    return fn(table)
